# baseline (speedup 1.0000x reference)
import functools

import jax
import jax.numpy as jnp
from jax import lax
from jax.experimental import pallas as pl
from jax.experimental.pallas import tpu as pltpu
from jax.experimental.pallas import tpu_sc as plsc

_VOCAB = 100000
_MAX_LEN = 2048
_DIM = 1024
_BATCH = 4

_NC = 2
_NS = 16
_NW = _NC * _NS
_T_PER_W = _MAX_LEN // _NW
_CHUNK = 16
_NCHUNK = _BATCH * _T_PER_W // _CHUNK
_NH = _T_PER_W // _CHUNK
_LANES = 16
_NBUF = 5
_GDEPTH = 3

_mesh = plsc.VectorSubcoreMesh(core_axis_name="c", subcore_axis_name="s")


@functools.partial(
    pl.kernel,
    mesh=_mesh,
    out_type=jax.ShapeDtypeStruct((_BATCH, _MAX_LEN, _DIM), jnp.float32),
    scratch_types=(
        [pltpu.VMEM((_BATCH * _T_PER_W,), jnp.int32)]
        + [pltpu.VMEM((_CHUNK * _DIM // 2,), jnp.int32) for _ in range(2)]
        + [pltpu.VMEM((_CHUNK, _DIM), jnp.float32) for _ in range(_NBUF)]
        + [pltpu.SemaphoreType.DMA for _ in range(3 + 2 * _NBUF)]
    ),
)
def _embed(idx_hbm, table_hbm, pos_hbm, out_hbm, idx_v, *scratch):
    pring = scratch[:2]
    bufs = scratch[2:2 + _NBUF]
    psems = scratch[2 + _NBUF:4 + _NBUF]
    isem = scratch[4 + _NBUF]
    gsems = scratch[5 + _NBUF:5 + 2 * _NBUF]
    wsems = scratch[5 + 2 * _NBUF:5 + 3 * _NBUF]

    wid = lax.axis_index("s") * _NC + lax.axis_index("c")
    t0 = wid * _T_PER_W

    idx_handles = [
        pltpu.async_copy(idx_hbm.at[b, pl.ds(t0, _T_PER_W)],
                         idx_v.at[pl.ds(b * _T_PER_W, _T_PER_W)], isem)
        for b in range(_BATCH)
    ]

    def pos_load(h):
        return pltpu.async_copy(
            pos_hbm.at[pl.ds((t0 + h * _CHUNK) * (_DIM // 2),
                             _CHUNK * _DIM // 2)],
            pring[h % 2], psems[h % 2])

    def gather(c):
        h, b = divmod(c, _BATCH)
        return pltpu.async_copy(
            table_hbm.at[idx_v.at[pl.ds(b * _T_PER_W + h * _CHUNK, _CHUNK)]],
            bufs[c % _NBUF], gsems[c % _NBUF])

    def writeback(c):
        h, b = divmod(c, _BATCH)
        return pltpu.async_copy(
            bufs[c % _NBUF],
            out_hbm.at[b, pl.ds(t0 + h * _CHUNK, _CHUNK)],
            wsems[c % _NBUF])

    hp = [pos_load(0), pos_load(1)]
    for hnd in idx_handles:
        hnd.wait()
    pos_ready = [False, False]
    hw = [None] * _NBUF
    hg = [None] * _NBUF
    issued = 0
    for c in range(_NCHUNK):
        h = c // _BATCH
        while issued < min(c + 1 + _GDEPTH, _NCHUNK):
            slot = issued % _NBUF
            if hw[slot] is not None:
                hw[slot].wait()
                hw[slot] = None
            hg[slot] = gather(issued)
            issued += 1
        hg[c % _NBUF].wait()
        if not pos_ready[h % 2]:
            hp[h % 2].wait()
            pos_ready[h % 2] = True
        buf = bufs[c % _NBUF]
        pos = pring[h % 2]

        def add_row(r, _, buf=buf, pos=pos):
            base = pl.multiple_of(r * (_DIM // 2), _DIM // 2)
            for g in range(_DIM // (2 * _LANES)):
                bits = pos[pl.ds(base + g * _LANES, _LANES)]
                a = lax.bitcast_convert_type(bits << 16, jnp.float32)
                b2 = lax.bitcast_convert_type(
                    bits & jnp.int32(-65536), jnp.float32)
                plsc.addupdate(buf.at[r, pl.ds(g * 2 * _LANES, _LANES)], a)
                plsc.addupdate(
                    buf.at[r, pl.ds(g * 2 * _LANES + _LANES, _LANES)], b2)
            return 0

        lax.fori_loop(0, _CHUNK, add_row, 0)
        if c % _BATCH == _BATCH - 1:
            pos_ready[h % 2] = False
            if h + 2 <= _NH - 1:
                hp[h % 2] = pos_load(h + 2)
        hw[c % _NBUF] = writeback(c)
    for hnd in hw:
        if hnd is not None:
            hnd.wait()


def kernel(inputs, token_table, pos_table):
    p = pos_table.reshape(_MAX_LEN, _DIM // (2 * _LANES), 2, _LANES)
    pairs = jnp.stack(
        [p[:, :, 0, :].astype(jnp.bfloat16),
         p[:, :, 1, :].astype(jnp.bfloat16)], axis=-1)
    pos_i32 = lax.bitcast_convert_type(pairs, jnp.int32)
    pos_i32 = pos_i32.reshape(_MAX_LEN * _DIM // 2)
    return _embed(inputs.astype(jnp.int32), token_table, pos_i32)


# speedup vs baseline: 1.8630x; 1.8630x over previous
import functools

import jax
import jax.numpy as jnp
from jax import lax
from jax.experimental import pallas as pl
from jax.experimental.pallas import tpu as pltpu
from jax.experimental.pallas import tpu_sc as plsc

_VOCAB = 100000
_MAX_LEN = 2048
_DIM = 1024
_BATCH = 4

_NC = 2
_NS = 16
_NW = _NC * _NS
_T_PER_W = _MAX_LEN // _NW
_CHUNK = 16
_NCHUNK = _BATCH * _T_PER_W // _CHUNK
_NH = _T_PER_W // _CHUNK
_LANES = 16
_NBUF = 5
_GDEPTH = 3

_mesh = plsc.VectorSubcoreMesh(core_axis_name="c", subcore_axis_name="s")


@functools.partial(
    pl.kernel,
    mesh=_mesh,
    out_type=jax.ShapeDtypeStruct((_BATCH, _MAX_LEN, _DIM), jnp.float32),
    scratch_types=(
        [pltpu.VMEM((_BATCH * _T_PER_W,), jnp.int32)]
        + [pltpu.VMEM((_CHUNK, _DIM), jnp.float32) for _ in range(2)]
        + [pltpu.VMEM((_CHUNK, _DIM), jnp.float32) for _ in range(_NBUF)]
        + [pltpu.SemaphoreType.DMA for _ in range(3 + 2 * _NBUF)]
    ),
)
def _embed(idx_hbm, table_hbm, pos_hbm, out_hbm, idx_v, *scratch):
    pring = scratch[:2]
    bufs = scratch[2:2 + _NBUF]
    psems = scratch[2 + _NBUF:4 + _NBUF]
    isem = scratch[4 + _NBUF]
    gsems = scratch[5 + _NBUF:5 + 2 * _NBUF]
    wsems = scratch[5 + 2 * _NBUF:5 + 3 * _NBUF]

    wid = lax.axis_index("s") * _NC + lax.axis_index("c")
    t0 = wid * _T_PER_W

    idx_handles = [
        pltpu.async_copy(idx_hbm.at[b, pl.ds(t0, _T_PER_W)],
                         idx_v.at[pl.ds(b * _T_PER_W, _T_PER_W)], isem)
        for b in range(_BATCH)
    ]

    def pos_load(h):
        return pltpu.async_copy(
            pos_hbm.at[pl.ds(t0 + h * _CHUNK, _CHUNK)],
            pring[h % 2], psems[h % 2])

    def gather(c):
        h, b = divmod(c, _BATCH)
        return pltpu.async_copy(
            table_hbm.at[idx_v.at[pl.ds(b * _T_PER_W + h * _CHUNK, _CHUNK)]],
            bufs[c % _NBUF], gsems[c % _NBUF])

    def writeback_half(c, half):
        h, b = divmod(c, _BATCH)
        hc = _CHUNK // 2
        return pltpu.async_copy(
            bufs[c % _NBUF].at[pl.ds(half * hc, hc)],
            out_hbm.at[b, pl.ds(t0 + h * _CHUNK + half * hc, hc)],
            wsems[c % _NBUF])

    hp = [pos_load(0), pos_load(1)]
    for hnd in idx_handles:
        hnd.wait()
    pos_ready = [False, False]
    hw = [None] * _NBUF
    hg = [None] * _NBUF
    issued = 0
    for c in range(_NCHUNK):
        h = c // _BATCH
        while issued < min(c + 1 + _GDEPTH, _NCHUNK):
            slot = issued % _NBUF
            if hw[slot] is not None:
                for hnd in hw[slot]:
                    hnd.wait()
                hw[slot] = None
            hg[slot] = gather(issued)
            issued += 1
        hg[c % _NBUF].wait()
        if not pos_ready[h % 2]:
            hp[h % 2].wait()
            pos_ready[h % 2] = True
        buf = bufs[c % _NBUF]
        pos = pring[h % 2]

        def add_half_row(i, _, buf=buf, pos=pos):
            r = i // 2
            col0 = (i % 2) * (_DIM // 2)
            for cc in range(_DIM // (2 * _LANES)):
                sl = pl.ds(col0 + cc * _LANES, _LANES)
                plsc.addupdate(buf.at[r, sl], pos[r, sl])
            return 0

        lax.fori_loop(0, _CHUNK, add_half_row, 0)
        hw0 = writeback_half(c, 0)
        lax.fori_loop(_CHUNK, 2 * _CHUNK, add_half_row, 0)
        hw1 = writeback_half(c, 1)
        if c % _BATCH == _BATCH - 1:
            pos_ready[h % 2] = False
            if h + 2 <= _NH - 1:
                hp[h % 2] = pos_load(h + 2)
        hw[c % _NBUF] = (hw0, hw1)
    for pair in hw:
        if pair is not None:
            for hnd in pair:
                hnd.wait()


def kernel(inputs, token_table, pos_table):
    return _embed(inputs.astype(jnp.int32), token_table, pos_table)


# speedup vs baseline: 2.1355x; 1.1463x over previous
import functools

import jax
import jax.numpy as jnp
from jax import lax
from jax.experimental import pallas as pl
from jax.experimental.pallas import tpu as pltpu
from jax.experimental.pallas import tpu_sc as plsc

_VOCAB = 100000
_MAX_LEN = 2048
_DIM = 1024
_BATCH = 4

_NC = 2
_NS = 16
_NW = _NC * _NS
_T_PER_W = _MAX_LEN // _NW
_CHUNK = 8
_NG = _T_PER_W // _CHUNK
_GBUF = 3
_LANES = 16

_mesh = plsc.VectorSubcoreMesh(core_axis_name="c", subcore_axis_name="s")


@functools.partial(
    pl.kernel,
    mesh=_mesh,
    out_type=jax.ShapeDtypeStruct((_BATCH, _MAX_LEN, _DIM), jnp.float32),
    scratch_types=(
        [pltpu.VMEM((_BATCH * _T_PER_W,), jnp.int32)]
        + [pltpu.VMEM((_CHUNK, _DIM), jnp.float32) for _ in range(2)]
        + [pltpu.VMEM((_CHUNK, _DIM), jnp.float32)
           for _ in range(_GBUF * _BATCH)]
        + [pltpu.SemaphoreType.DMA for _ in range(3 + 2 * _GBUF * _BATCH)]
    ),
)
def _embed(idx_hbm, table_hbm, pos_hbm, out_hbm, idx_v, *scratch):
    nslots = _GBUF * _BATCH
    pring = scratch[:2]
    bufs = scratch[2:2 + nslots]
    psems = scratch[2 + nslots:4 + nslots]
    isem = scratch[4 + nslots]
    gsems = scratch[5 + nslots:5 + 2 * nslots]
    wsems = scratch[5 + 2 * nslots:5 + 3 * nslots]

    wid = lax.axis_index("s") * _NC + lax.axis_index("c")
    t0 = wid * _T_PER_W

    idx_handles = [
        pltpu.async_copy(idx_hbm.at[b, pl.ds(t0, _T_PER_W)],
                         idx_v.at[pl.ds(b * _T_PER_W, _T_PER_W)], isem)
        for b in range(_BATCH)
    ]

    def pos_load(g):
        return pltpu.async_copy(
            pos_hbm.at[pl.ds(t0 + g * _CHUNK, _CHUNK)],
            pring[g % 2], psems[g % 2])

    def gather(g, j):
        slot = (g % _GBUF) * _BATCH + j
        return pltpu.async_copy(
            table_hbm.at[idx_v.at[pl.ds(j * _T_PER_W + g * _CHUNK, _CHUNK)]],
            bufs[slot], gsems[slot])

    def writeback(g, j):
        slot = (g % _GBUF) * _BATCH + j
        return pltpu.async_copy(
            bufs[slot],
            out_hbm.at[j, pl.ds(t0 + g * _CHUNK, _CHUNK)],
            wsems[slot])

    hp = [pos_load(0), pos_load(1)]
    for hnd in idx_handles:
        hnd.wait()
    pos_ready = [False, False]
    hw = [None] * _GBUF
    hg = [None] * _GBUF
    issued = 0
    for g in range(_NG):
        while issued < min(g + _GBUF, _NG):
            ring = issued % _GBUF
            if hw[ring] is not None:
                for hnd in hw[ring]:
                    hnd.wait()
                hw[ring] = None
            hg[ring] = [gather(issued, j) for j in range(_BATCH)]
            issued += 1
        for hnd in hg[g % _GBUF]:
            hnd.wait()
        if not pos_ready[g % 2]:
            hp[g % 2].wait()
            pos_ready[g % 2] = True
        gbufs = bufs[(g % _GBUF) * _BATCH:(g % _GBUF) * _BATCH + _BATCH]
        pos = pring[g % 2]

        def add_row(r, _, gbufs=gbufs, pos=pos):
            for cc in range(_DIM // _LANES):
                sl = pl.ds(cc * _LANES, _LANES)
                v = pos[r, sl]
                for bb in gbufs:
                    plsc.addupdate(bb.at[r, sl], v)
            return 0

        lax.fori_loop(0, _CHUNK, add_row, 0)
        pos_ready[g % 2] = False
        if g + 2 <= _NG - 1:
            hp[g % 2] = pos_load(g + 2)
        hw[g % _GBUF] = [writeback(g, j) for j in range(_BATCH)]
    for group in hw:
        if group is not None:
            for hnd in group:
                hnd.wait()


def kernel(inputs, token_table, pos_table):
    return _embed(inputs.astype(jnp.int32), token_table, pos_table)


# speedup vs baseline: 2.2039x; 1.0320x over previous
import functools

import jax
import jax.numpy as jnp
from jax import lax
from jax.experimental import pallas as pl
from jax.experimental.pallas import tpu as pltpu
from jax.experimental.pallas import tpu_sc as plsc

_VOCAB = 100000
_MAX_LEN = 2048
_DIM = 1024
_BATCH = 4

_NC = 2
_NS = 16
_NW = _NC * _NS
_T_PER_W = _MAX_LEN // _NW
_CHUNK = 16
_NCHUNK = _BATCH * _T_PER_W // _CHUNK
_NH = _T_PER_W // _CHUNK
_LANES = 16
_NBUF = 5
_GDEPTH = 3

_mesh = plsc.VectorSubcoreMesh(core_axis_name="c", subcore_axis_name="s")


@functools.partial(
    pl.kernel,
    mesh=_mesh,
    out_type=jax.ShapeDtypeStruct((_BATCH, _MAX_LEN, _DIM), jnp.float32),
    scratch_types=(
        [pltpu.VMEM((_BATCH * _T_PER_W,), jnp.int32)]
        + [pltpu.VMEM((_CHUNK, _DIM), jnp.float32) for _ in range(2)]
        + [pltpu.VMEM((_CHUNK, _DIM), jnp.float32) for _ in range(_NBUF)]
        + [pltpu.SemaphoreType.DMA for _ in range(3 + 2 * _NBUF)]
    ),
)
def _embed(idx_hbm, table_hbm, pos_hbm, out_hbm, idx_v, *scratch):
    pring = scratch[:2]
    bufs = scratch[2:2 + _NBUF]
    psems = scratch[2 + _NBUF:4 + _NBUF]
    isem = scratch[4 + _NBUF]
    gsems = scratch[5 + _NBUF:5 + 2 * _NBUF]
    wsems = scratch[5 + 2 * _NBUF:5 + 3 * _NBUF]

    wid = lax.axis_index("s") * _NC + lax.axis_index("c")
    t0 = wid * _T_PER_W

    idx_handles = [
        pltpu.async_copy(idx_hbm.at[b, pl.ds(t0, _T_PER_W)],
                         idx_v.at[pl.ds(b * _T_PER_W, _T_PER_W)], isem)
        for b in range(_BATCH)
    ]

    def pos_load(h):
        return pltpu.async_copy(
            pos_hbm.at[pl.ds(t0 + h * _CHUNK, _CHUNK)],
            pring[h % 2], psems[h % 2])

    def gather(c):
        h, b = divmod(c, _BATCH)
        return pltpu.async_copy(
            table_hbm.at[idx_v.at[pl.ds(b * _T_PER_W + h * _CHUNK, _CHUNK)]],
            bufs[c % _NBUF], gsems[c % _NBUF])

    def writeback(c):
        h, b = divmod(c, _BATCH)
        return pltpu.async_copy(
            bufs[c % _NBUF],
            out_hbm.at[b, pl.ds(t0 + h * _CHUNK, _CHUNK)],
            wsems[c % _NBUF])

    hp = [pos_load(0), pos_load(1)]
    for hnd in idx_handles:
        hnd.wait()
    pos_ready = [False, False]
    hw = [None] * _NBUF
    hg = [None] * _NBUF
    issued = 0
    for c in range(_NCHUNK):
        h = c // _BATCH
        while issued < min(c + 1 + _GDEPTH, _NCHUNK):
            slot = issued % _NBUF
            if hw[slot] is not None:
                hw[slot].wait()
                hw[slot] = None
            hg[slot] = gather(issued)
            issued += 1
        hg[c % _NBUF].wait()
        if not pos_ready[h % 2]:
            hp[h % 2].wait()
            pos_ready[h % 2] = True
        buf = bufs[c % _NBUF]
        pos = pring[h % 2]

        def add_row(r, _, buf=buf, pos=pos):
            for cc in range(_DIM // _LANES):
                sl = pl.ds(cc * _LANES, _LANES)
                plsc.addupdate(buf.at[r, sl], pos[r, sl])
            return 0

        lax.fori_loop(0, _CHUNK, add_row, 0)
        if c % _BATCH == _BATCH - 1:
            pos_ready[h % 2] = False
            if h + 2 <= _NH - 1:
                hp[h % 2] = pos_load(h + 2)
        hw[c % _NBUF] = writeback(c)
    for hnd in hw:
        if hnd is not None:
            hnd.wait()


def kernel(inputs, token_table, pos_table):
    return _embed(inputs.astype(jnp.int32), token_table, pos_table)


# speedup vs baseline: 2.2250x; 1.0096x over previous
import functools

import jax
import jax.numpy as jnp
from jax import lax
from jax.experimental import pallas as pl
from jax.experimental.pallas import tpu as pltpu
from jax.experimental.pallas import tpu_sc as plsc

_VOCAB = 100000
_MAX_LEN = 2048
_DIM = 1024
_BATCH = 4

_NC = 2
_NS = 16
_NW = _NC * _NS
_T_PER_W = _MAX_LEN // _NW
_CHUNK = 16
_NCHUNK = _BATCH * _T_PER_W // _CHUNK
_NH = _T_PER_W // _CHUNK
_LANES = 16
_NBUF = 5
_GDEPTH = 3

_mesh = plsc.VectorSubcoreMesh(core_axis_name="c", subcore_axis_name="s")


@functools.partial(
    pl.kernel,
    mesh=_mesh,
    out_type=jax.ShapeDtypeStruct((_BATCH, _MAX_LEN, _DIM), jnp.float32),
    scratch_types=(
        [pltpu.VMEM((_BATCH * _T_PER_W,), jnp.int32)]
        + [pltpu.VMEM((_CHUNK, _DIM), jnp.float32) for _ in range(2)]
        + [pltpu.VMEM((_CHUNK, _DIM), jnp.float32) for _ in range(_NBUF)]
        + [pltpu.SemaphoreType.DMA for _ in range(3 + 2 * _NBUF)]
    ),
)
def _embed(idx_hbm, table_hbm, pos_hbm, out_hbm, idx_v, *scratch):
    pring = scratch[:2]
    bufs = scratch[2:2 + _NBUF]
    psems = scratch[2 + _NBUF:4 + _NBUF]
    isem = scratch[4 + _NBUF]
    gsems = scratch[5 + _NBUF:5 + 2 * _NBUF]
    wsems = scratch[5 + 2 * _NBUF:5 + 3 * _NBUF]

    wid = lax.axis_index("s") * _NC + lax.axis_index("c")
    t0 = wid * _T_PER_W

    idx_handles = [
        pltpu.async_copy(idx_hbm.at[b, pl.ds(t0, _T_PER_W)],
                         idx_v.at[pl.ds(b * _T_PER_W, _T_PER_W)], isem)
        for b in range(_BATCH)
    ]

    def pos_load(h):
        return pltpu.async_copy(
            pos_hbm.at[pl.ds(t0 + h * _CHUNK, _CHUNK)],
            pring[h % 2], psems[h % 2])

    def gather(c):
        h, b = divmod(c, _BATCH)
        return pltpu.async_copy(
            table_hbm.at[idx_v.at[pl.ds(b * _T_PER_W + h * _CHUNK, _CHUNK)]],
            bufs[c % _NBUF], gsems[c % _NBUF])

    def writeback(c):
        h, b = divmod(c, _BATCH)
        return pltpu.async_copy(
            bufs[c % _NBUF],
            out_hbm.at[b, pl.ds(t0 + h * _CHUNK, _CHUNK)],
            wsems[c % _NBUF])

    hp = [pos_load(0), pos_load(1)]
    for hnd in idx_handles:
        hnd.wait()
    pos_ready = [False, False]
    hw = [None] * _NBUF
    hg = [None] * _NBUF
    issued = 0
    for c in range(_NCHUNK):
        h = c // _BATCH
        while issued < min(c + 1 + _GDEPTH, _NCHUNK):
            slot = issued % _NBUF
            if hw[slot] is not None:
                hw[slot].wait()
                hw[slot] = None
            hg[slot] = gather(issued)
            issued += 1
        hg[c % _NBUF].wait()
        if not pos_ready[h % 2]:
            hp[h % 2].wait()
            pos_ready[h % 2] = True
        if c % 2 == 1:
            buf0 = bufs[(c - 1) % _NBUF]
            buf1 = bufs[c % _NBUF]
            pos = pring[h % 2]

            def add_row(r, _, buf0=buf0, buf1=buf1, pos=pos):
                for cc in range(_DIM // _LANES):
                    sl = pl.ds(cc * _LANES, _LANES)
                    v = pos[r, sl]
                    plsc.addupdate(buf0.at[r, sl], v)
                    plsc.addupdate(buf1.at[r, sl], v)
                return 0

            lax.fori_loop(0, _CHUNK, add_row, 0)
            hw[(c - 1) % _NBUF] = writeback(c - 1)
            hw[c % _NBUF] = writeback(c)
        if c % _BATCH == _BATCH - 1:
            pos_ready[h % 2] = False
            if h + 2 <= _NH - 1:
                hp[h % 2] = pos_load(h + 2)
    for hnd in hw:
        if hnd is not None:
            hnd.wait()


def kernel(inputs, token_table, pos_table):
    return _embed(inputs.astype(jnp.int32), token_table, pos_table)
